# Initial kernel scaffold; baseline (speedup 1.0000x reference)
import jax
import jax.numpy as jnp
from jax.experimental import pallas as pl


def kernel(x, edge_index, edge_attr):
    raise NotImplementedError("write your pallas kernel here")


# speedup vs baseline: 4.7128x; 4.7128x over previous
import functools

import jax
import jax.numpy as jnp
from jax import lax
from jax.experimental import pallas as pl
from jax.experimental.pallas import tpu as pltpu
from jax.experimental.pallas import tpu_sc as plsc

N_NODES = 10000
N_EDGES = 320000
D_FEAT = 128

NC = 2
NS = 16
NW = NC * NS
E_PER_W = N_EDGES // NW
CHUNK = 2000
N_CHUNKS = E_PER_W // CHUNK
NODES_PAD = 10240
X_PER_W = NODES_PAD // NS


def _sc_body(xflat_hbm, edge_hbm, attr_hbm, out_hbm,
             xidx_v, xval_v, x0_sh, x0_v, src_v, dst_v, attr_v,
             acc_sx, acc_cx, acc_sy, acc_cy, sem):
    cid = lax.axis_index("c")
    sid = lax.axis_index("s")
    wid = cid * NS + sid

    lanes = lax.iota(jnp.int32, 16)
    zf = jnp.zeros((16,), jnp.float32)
    onef = jnp.full((16,), 1.0, jnp.float32)
    col0 = jnp.zeros((16,), jnp.int32)
    col1 = jnp.full((16,), 1, jnp.int32)

    nb = sid * X_PER_W

    def build_idx(j, carry):
        node = jnp.minimum(nb + j * 16 + lanes, N_NODES - 1)
        xidx_v[pl.ds(j * 16, 16)] = node * D_FEAT
        return carry

    lax.fori_loop(0, X_PER_W // 16, build_idx, 0)
    pltpu.async_copy(xflat_hbm.at[xidx_v], xval_v, sem).wait()
    pltpu.sync_copy(xval_v, x0_sh.at[pl.ds(nb, X_PER_W)])

    def zero_body(j, carry):
        acc_sx[pl.ds(j * 16, 16)] = zf
        acc_cx[pl.ds(j * 16, 16)] = zf
        acc_sy[pl.ds(j * 16, 16)] = zf
        acc_cy[pl.ds(j * 16, 16)] = zf
        return carry

    lax.fori_loop(0, NODES_PAD // 16, zero_body, 0)

    plsc.subcore_barrier()
    pltpu.sync_copy(x0_sh, x0_v)

    ebase = wid * E_PER_W

    def chunk_body(k, carry):
        gb = ebase + k * CHUNK
        pltpu.sync_copy(edge_hbm.at[pl.ds(gb, CHUNK)], src_v)
        pltpu.sync_copy(edge_hbm.at[pl.ds(N_EDGES + gb, CHUNK)], dst_v)
        pltpu.sync_copy(attr_hbm.at[pl.ds(gb * 4, CHUNK * 4)], attr_v)

        def body(i, c2):
            s = src_v[pl.ds(i * 16, 16)]
            d = dst_v[pl.ds(i * 16, 16)]
            xs = plsc.load_gather(x0_v, [s])
            xd = plsc.load_gather(x0_v, [d])
            idx4 = (i * 16 + lanes) * 4
            a0 = plsc.load_gather(attr_v, [idx4])
            a1 = plsc.load_gather(attr_v, [idx4 + col1])
            diff = xd - xs
            m0 = a0 != 0.0
            m1 = a1 != 0.0
            per0 = jnp.where(m0, diff / jnp.where(m0, a0, onef), zf)
            per1 = jnp.where(m1, diff / jnp.where(m1, a1, onef), zf)
            cnt0 = jnp.where(m0, onef, zf)
            cnt1 = jnp.where(m1, onef, zf)
            plsc.addupdate_scatter(acc_sx, [s], per0)
            plsc.addupdate_scatter(acc_cx, [s], cnt0)
            plsc.addupdate_scatter(acc_sy, [s], per1)
            plsc.addupdate_scatter(acc_cy, [s], cnt1)
            return c2

        lax.fori_loop(0, CHUNK // 16, body, 0)
        return carry

    lax.fori_loop(0, N_CHUNKS, chunk_body, 0)

    ob = wid * 4 * NODES_PAD
    pltpu.sync_copy(acc_sx, out_hbm.at[pl.ds(ob, NODES_PAD)])
    pltpu.sync_copy(acc_cx, out_hbm.at[pl.ds(ob + NODES_PAD, NODES_PAD)])
    pltpu.sync_copy(acc_sy, out_hbm.at[pl.ds(ob + 2 * NODES_PAD, NODES_PAD)])
    pltpu.sync_copy(acc_cy, out_hbm.at[pl.ds(ob + 3 * NODES_PAD, NODES_PAD)])


_sc_partials = functools.partial(
    pl.kernel,
    mesh=plsc.VectorSubcoreMesh(core_axis_name="c", subcore_axis_name="s"),
    compiler_params=pltpu.CompilerParams(needs_layout_passes=False),
    out_type=jax.ShapeDtypeStruct((NW * 4 * NODES_PAD,), jnp.float32),
    scratch_types=[
        pltpu.VMEM((X_PER_W,), jnp.int32),
        pltpu.VMEM((X_PER_W,), jnp.float32),
        pltpu.VMEM_SHARED((NODES_PAD,), jnp.float32),
        pltpu.VMEM((NODES_PAD,), jnp.float32),
        pltpu.VMEM((CHUNK,), jnp.int32),
        pltpu.VMEM((CHUNK,), jnp.int32),
        pltpu.VMEM((CHUNK * 4,), jnp.float32),
        pltpu.VMEM((NODES_PAD,), jnp.float32),
        pltpu.VMEM((NODES_PAD,), jnp.float32),
        pltpu.VMEM((NODES_PAD,), jnp.float32),
        pltpu.VMEM((NODES_PAD,), jnp.float32),
        pltpu.SemaphoreType.DMA,
    ],
)(_sc_body)


def _tc_reduce(parts_ref, out_ref):
    p = parts_ref[...].reshape(NW, 4, NODES_PAD)
    s = jnp.sum(p, axis=0)
    dx = s[0:1, :] / jnp.maximum(s[1:2, :], 1.0)
    dy = s[2:3, :] / jnp.maximum(s[3:4, :], 1.0)
    out_ref[0:1, :] = dx
    out_ref[1:2, :] = dy


def kernel(x, edge_index, edge_attr):
    xflat = x.reshape(-1)
    parts = _sc_partials(xflat, edge_index.reshape(-1), edge_attr.reshape(-1))
    out2 = pl.pallas_call(
        _tc_reduce,
        out_shape=jax.ShapeDtypeStruct((2, NODES_PAD), jnp.float32),
    )(parts)
    return out2[:, :N_NODES].T


# speedup vs baseline: 13.8943x; 2.9482x over previous
import functools

import jax
import jax.numpy as jnp
from jax import lax
from jax.experimental import pallas as pl
from jax.experimental.pallas import tpu as pltpu
from jax.experimental.pallas import tpu_sc as plsc

N_NODES = 10000
N_EDGES = 320000
D_FEAT = 128

NC = 2
NS = 16
NW = NC * NS
CHUNK = 2048
N_CHUNKS = -(-N_EDGES // CHUNK)
TAIL = N_EDGES - (N_CHUNKS - 1) * CHUNK
NODES_PAD = 10240
X_ROWS = 320


def _sc_body(x_hbm, src_hbm, dst_hbm, ax_hbm, ay_hbm, out_hbm,
             xbuf_v, x0_sh, x0_v, src_v, dst_v, ax_v, ay_v,
             acc_sx, acc_cx, acc_sy, acc_cy, sem):
    cid = lax.axis_index("c")
    sid = lax.axis_index("s")
    wid = cid * NS + sid

    lanes = lax.iota(jnp.int32, 16)
    zf = jnp.zeros((16,), jnp.float32)
    onef = jnp.full((16,), 1.0, jnp.float32)
    col0 = jnp.zeros((16,), jnp.int32)

    for b in range(2):
        nb = sid * (2 * X_ROWS) + b * X_ROWS
        start = jnp.minimum(nb, N_NODES - X_ROWS)
        pltpu.sync_copy(x_hbm.at[pl.ds(start, X_ROWS), :], xbuf_v)

        def extract(j, carry):
            node = jnp.minimum(nb + j * 16 + lanes, N_NODES - 1)
            vals = plsc.load_gather(xbuf_v, [node - start, col0])
            x0_v[pl.ds(b * X_ROWS + j * 16, 16)] = vals
            return carry

        lax.fori_loop(0, X_ROWS // 16, extract, 0)
    pltpu.sync_copy(x0_v.at[pl.ds(0, 2 * X_ROWS)],
                    x0_sh.at[pl.ds(sid * 2 * X_ROWS, 2 * X_ROWS)])

    def zero_body(j, carry):
        acc_sx[pl.ds(j * 16, 16)] = zf
        acc_cx[pl.ds(j * 16, 16)] = zf
        acc_sy[pl.ds(j * 16, 16)] = zf
        acc_cy[pl.ds(j * 16, 16)] = zf
        return carry

    lax.fori_loop(0, NODES_PAD // 16, zero_body, 0)

    plsc.subcore_barrier()
    pltpu.sync_copy(x0_sh, x0_v)

    def do_chunk(c, n_edges_static):
        gb = c * CHUNK
        pltpu.sync_copy(src_hbm.at[pl.ds(gb, n_edges_static)],
                        src_v.at[pl.ds(0, n_edges_static)])
        pltpu.sync_copy(dst_hbm.at[pl.ds(gb, n_edges_static)],
                        dst_v.at[pl.ds(0, n_edges_static)])
        pltpu.sync_copy(ax_hbm.at[pl.ds(gb, n_edges_static)],
                        ax_v.at[pl.ds(0, n_edges_static)])
        pltpu.sync_copy(ay_hbm.at[pl.ds(gb, n_edges_static)],
                        ay_v.at[pl.ds(0, n_edges_static)])

        def inner(i, c2):
            s = src_v[pl.ds(i * 16, 16)]
            d = dst_v[pl.ds(i * 16, 16)]
            xs = plsc.load_gather(x0_v, [s])
            xd = plsc.load_gather(x0_v, [d])
            a0 = ax_v[pl.ds(i * 16, 16)]
            a1 = ay_v[pl.ds(i * 16, 16)]
            diff = xd - xs
            m0 = a0 != 0.0
            m1 = a1 != 0.0
            per0 = jnp.where(m0, diff / jnp.where(m0, a0, onef), zf)
            per1 = jnp.where(m1, diff / jnp.where(m1, a1, onef), zf)
            cnt0 = jnp.where(m0, onef, zf)
            cnt1 = jnp.where(m1, onef, zf)
            plsc.addupdate_scatter(acc_sx, [s], per0)
            plsc.addupdate_scatter(acc_cx, [s], cnt0)
            plsc.addupdate_scatter(acc_sy, [s], per1)
            plsc.addupdate_scatter(acc_cy, [s], cnt1)
            return c2

        lax.fori_loop(0, n_edges_static // 16, inner, 0)

    for k in range(-(-N_CHUNKS // NW)):
        c = wid + k * NW

        @pl.when(c < N_CHUNKS - 1)
        def _full():
            do_chunk(c, CHUNK)

        @pl.when(c == N_CHUNKS - 1)
        def _tail():
            do_chunk(c, TAIL)

    ob = wid * 4 * NODES_PAD
    pltpu.sync_copy(acc_sx, out_hbm.at[pl.ds(ob, NODES_PAD)])
    pltpu.sync_copy(acc_cx, out_hbm.at[pl.ds(ob + NODES_PAD, NODES_PAD)])
    pltpu.sync_copy(acc_sy, out_hbm.at[pl.ds(ob + 2 * NODES_PAD, NODES_PAD)])
    pltpu.sync_copy(acc_cy, out_hbm.at[pl.ds(ob + 3 * NODES_PAD, NODES_PAD)])


_sc_partials = functools.partial(
    pl.kernel,
    mesh=plsc.VectorSubcoreMesh(core_axis_name="c", subcore_axis_name="s"),
    compiler_params=pltpu.CompilerParams(needs_layout_passes=False),
    out_type=jax.ShapeDtypeStruct((NW * 4 * NODES_PAD,), jnp.float32),
    scratch_types=[
        pltpu.VMEM((X_ROWS, D_FEAT), jnp.float32),
        pltpu.VMEM_SHARED((NODES_PAD,), jnp.float32),
        pltpu.VMEM((NODES_PAD,), jnp.float32),
        pltpu.VMEM((CHUNK,), jnp.int32),
        pltpu.VMEM((CHUNK,), jnp.int32),
        pltpu.VMEM((CHUNK,), jnp.float32),
        pltpu.VMEM((CHUNK,), jnp.float32),
        pltpu.VMEM((NODES_PAD,), jnp.float32),
        pltpu.VMEM((NODES_PAD,), jnp.float32),
        pltpu.VMEM((NODES_PAD,), jnp.float32),
        pltpu.VMEM((NODES_PAD,), jnp.float32),
        pltpu.SemaphoreType.DMA,
    ],
)(_sc_body)


def _tc_reduce(parts_ref, out_ref):
    p = parts_ref[...].reshape(NW, 4, NODES_PAD)
    s = jnp.sum(p, axis=0)
    dx = s[0:1, :] / jnp.maximum(s[1:2, :], 1.0)
    dy = s[2:3, :] / jnp.maximum(s[3:4, :], 1.0)
    out_ref[0:1, :] = dx
    out_ref[1:2, :] = dy


def kernel(x, edge_index, edge_attr):
    src = edge_index[0]
    dst = edge_index[1]
    ax = edge_attr[:, 0]
    ay = edge_attr[:, 1]
    parts = _sc_partials(x, src, dst, ax, ay)
    out2 = pl.pallas_call(
        _tc_reduce,
        out_shape=jax.ShapeDtypeStruct((2, NODES_PAD), jnp.float32),
    )(parts)
    return out2[:, :N_NODES].T


# speedup vs baseline: 16.9712x; 1.2215x over previous
import functools

import jax
import jax.numpy as jnp
from jax import lax
from jax.experimental import pallas as pl
from jax.experimental.pallas import tpu as pltpu
from jax.experimental.pallas import tpu_sc as plsc

N_NODES = 10000
N_EDGES = 320000
D_FEAT = 128

NC = 2
NS = 16
NW = NC * NS
CHUNK = 2048
N_CHUNKS = -(-N_EDGES // CHUNK)
TAIL = N_EDGES - (N_CHUNKS - 1) * CHUNK
NODES_PAD = 10240
X_ROWS = 320


def _sc_body(x_hbm, edge_hbm, ax_hbm, ay_hbm, out_hbm,
             xbuf_v, x0_sh, x0_v, edge_v, ax_v, ay_v,
             acc_sx, acc_cx, acc_sy, acc_cy, sem):
    cid = lax.axis_index("c")
    sid = lax.axis_index("s")
    wid = cid * NS + sid

    lanes = lax.iota(jnp.int32, 16)
    zf = jnp.zeros((16,), jnp.float32)
    onef = jnp.full((16,), 1.0, jnp.float32)
    col0 = jnp.zeros((16,), jnp.int32)

    for b in range(2):
        nb = sid * (2 * X_ROWS) + b * X_ROWS
        start = jnp.minimum(nb, N_NODES - X_ROWS)
        pltpu.sync_copy(x_hbm.at[pl.ds(start, X_ROWS), :], xbuf_v)

        def extract(j, carry):
            node = jnp.minimum(nb + j * 16 + lanes, N_NODES - 1)
            vals = plsc.load_gather(xbuf_v, [node - start, col0])
            x0_v[pl.ds(b * X_ROWS + j * 16, 16)] = vals
            return carry

        lax.fori_loop(0, X_ROWS // 16, extract, 0)
    pltpu.sync_copy(x0_v.at[pl.ds(0, 2 * X_ROWS)],
                    x0_sh.at[pl.ds(sid * 2 * X_ROWS, 2 * X_ROWS)])

    def zero_body(j, carry):
        acc_sx[pl.ds(j * 16, 16)] = zf
        acc_cx[pl.ds(j * 16, 16)] = zf
        acc_sy[pl.ds(j * 16, 16)] = zf
        acc_cy[pl.ds(j * 16, 16)] = zf
        return carry

    lax.fori_loop(0, NODES_PAD // 16, zero_body, 0)

    plsc.subcore_barrier()
    pltpu.sync_copy(x0_sh, x0_v)

    def do_chunk(c, n_edges_static):
        gb = c * CHUNK
        pltpu.sync_copy(edge_hbm.at[:, pl.ds(gb, n_edges_static)],
                        edge_v.at[:, pl.ds(0, n_edges_static)])
        pltpu.sync_copy(ax_hbm.at[pl.ds(gb, n_edges_static)],
                        ax_v.at[pl.ds(0, n_edges_static)])
        pltpu.sync_copy(ay_hbm.at[pl.ds(gb, n_edges_static)],
                        ay_v.at[pl.ds(0, n_edges_static)])

        def inner(i, c2):
            s = edge_v[0, pl.ds(i * 16, 16)]
            d = edge_v[1, pl.ds(i * 16, 16)]
            xs = plsc.load_gather(x0_v, [s])
            xd = plsc.load_gather(x0_v, [d])
            a0 = ax_v[pl.ds(i * 16, 16)]
            a1 = ay_v[pl.ds(i * 16, 16)]
            diff = xd - xs
            m0 = a0 != 0.0
            m1 = a1 != 0.0
            per0 = jnp.where(m0, diff / jnp.where(m0, a0, onef), zf)
            per1 = jnp.where(m1, diff / jnp.where(m1, a1, onef), zf)
            cnt0 = jnp.where(m0, onef, zf)
            cnt1 = jnp.where(m1, onef, zf)
            plsc.addupdate_scatter(acc_sx, [s], per0)
            plsc.addupdate_scatter(acc_cx, [s], cnt0)
            plsc.addupdate_scatter(acc_sy, [s], per1)
            plsc.addupdate_scatter(acc_cy, [s], cnt1)
            return c2

        lax.fori_loop(0, n_edges_static // 16, inner, 0)

    for k in range(-(-N_CHUNKS // NW)):
        c = wid + k * NW

        @pl.when(c < N_CHUNKS - 1)
        def _full():
            do_chunk(c, CHUNK)

        @pl.when(c == N_CHUNKS - 1)
        def _tail():
            do_chunk(c, TAIL)

    ob = wid * 4 * NODES_PAD
    pltpu.sync_copy(acc_sx, out_hbm.at[pl.ds(ob, NODES_PAD)])
    pltpu.sync_copy(acc_cx, out_hbm.at[pl.ds(ob + NODES_PAD, NODES_PAD)])
    pltpu.sync_copy(acc_sy, out_hbm.at[pl.ds(ob + 2 * NODES_PAD, NODES_PAD)])
    pltpu.sync_copy(acc_cy, out_hbm.at[pl.ds(ob + 3 * NODES_PAD, NODES_PAD)])


_sc_partials = functools.partial(
    pl.kernel,
    mesh=plsc.VectorSubcoreMesh(core_axis_name="c", subcore_axis_name="s"),
    compiler_params=pltpu.CompilerParams(needs_layout_passes=False),
    out_type=jax.ShapeDtypeStruct((NW * 4 * NODES_PAD,), jnp.float32),
    scratch_types=[
        pltpu.VMEM((X_ROWS, D_FEAT), jnp.float32),
        pltpu.VMEM_SHARED((NODES_PAD,), jnp.float32),
        pltpu.VMEM((NODES_PAD,), jnp.float32),
        pltpu.VMEM((2, CHUNK), jnp.int32),
        pltpu.VMEM((CHUNK,), jnp.float32),
        pltpu.VMEM((CHUNK,), jnp.float32),
        pltpu.VMEM((NODES_PAD,), jnp.float32),
        pltpu.VMEM((NODES_PAD,), jnp.float32),
        pltpu.VMEM((NODES_PAD,), jnp.float32),
        pltpu.VMEM((NODES_PAD,), jnp.float32),
        pltpu.SemaphoreType.DMA,
    ],
)(_sc_body)


def _tc_reduce(parts_ref, out_ref):
    p = parts_ref[...].reshape(NW, 4, NODES_PAD)
    s = jnp.sum(p, axis=0)
    dx = s[0:1, :] / jnp.maximum(s[1:2, :], 1.0)
    dy = s[2:3, :] / jnp.maximum(s[3:4, :], 1.0)
    out_ref[0:1, :] = dx
    out_ref[1:2, :] = dy


def kernel(x, edge_index, edge_attr):
    ax = edge_attr[:, 0]
    ay = edge_attr[:, 1]
    parts = _sc_partials(x, edge_index, ax, ay)
    out2 = pl.pallas_call(
        _tc_reduce,
        out_shape=jax.ShapeDtypeStruct((2, NODES_PAD), jnp.float32),
    )(parts)
    return out2[:, :N_NODES].T


# speedup vs baseline: 19.2643x; 1.1351x over previous
import functools

import jax
import jax.numpy as jnp
from jax import lax
from jax.experimental import pallas as pl
from jax.experimental.pallas import tpu as pltpu
from jax.experimental.pallas import tpu_sc as plsc

N_NODES = 10000
N_EDGES = 320000
D_FEAT = 128

NC = 2
NS = 16
NW = NC * NS
CHUNK = 2048
N_CHUNKS = -(-N_EDGES // CHUNK)
N_FULL = N_CHUNKS - 1
TAIL = N_EDGES - N_FULL * CHUNK
TAIL_WID = N_FULL % NW
SLOTS = -(-N_CHUNKS // NW)
NODES_PAD = 10240
X_ROWS = 320


def _sc_body(x_hbm, edge_hbm, ax_hbm, ay_hbm, out_hbm,
             xbuf_v, x0_sh, x0_v, edge_v, ax_v, ay_v,
             acc_sx, acc_cx, acc_sy, acc_cy, sem0, sem1):
    cid = lax.axis_index("c")
    sid = lax.axis_index("s")
    wid = cid * NS + sid
    sems = (sem0, sem1)

    lanes = lax.iota(jnp.int32, 16)
    zf = jnp.zeros((16,), jnp.float32)
    onef = jnp.full((16,), 1.0, jnp.float32)
    col0 = jnp.zeros((16,), jnp.int32)

    def chunk_copies(k, b):
        gb = (wid + k * NW) * CHUNK
        return (
            pltpu.make_async_copy(edge_hbm.at[:, pl.ds(gb, CHUNK)],
                                  edge_v.at[b], sems[b]),
            pltpu.make_async_copy(ax_hbm.at[pl.ds(gb, CHUNK)],
                                  ax_v.at[b], sems[b]),
            pltpu.make_async_copy(ay_hbm.at[pl.ds(gb, CHUNK)],
                                  ay_v.at[b], sems[b]),
        )

    def issue(k, b):
        @pl.when(wid + k * NW < N_FULL)
        def _():
            for cp in chunk_copies(k, b):
                cp.start()

    def wait(k, b):
        @pl.when(wid + k * NW < N_FULL)
        def _():
            for cp in chunk_copies(k, b):
                cp.wait()

    issue(0, 0)
    issue(1, 1)

    for blk in range(2):
        nb = sid * (2 * X_ROWS) + blk * X_ROWS
        start = jnp.minimum(nb, N_NODES - X_ROWS)
        pltpu.sync_copy(x_hbm.at[pl.ds(start, X_ROWS), :], xbuf_v)

        def extract(j, carry):
            node = jnp.minimum(nb + j * 16 + lanes, N_NODES - 1)
            vals = plsc.load_gather(xbuf_v, [node - start, col0])
            x0_v[pl.ds(blk * X_ROWS + j * 16, 16)] = vals
            return carry

        lax.fori_loop(0, X_ROWS // 16, extract, 0)
    pltpu.sync_copy(x0_v.at[pl.ds(0, 2 * X_ROWS)],
                    x0_sh.at[pl.ds(sid * 2 * X_ROWS, 2 * X_ROWS)])

    def zero_body(j, carry):
        acc_sx[pl.ds(j * 16, 16)] = zf
        acc_cx[pl.ds(j * 16, 16)] = zf
        acc_sy[pl.ds(j * 16, 16)] = zf
        acc_cy[pl.ds(j * 16, 16)] = zf
        return carry

    lax.fori_loop(0, NODES_PAD // 16, zero_body, 0)

    plsc.subcore_barrier()
    pltpu.sync_copy(x0_sh, x0_v)

    def edge_group(b, i):
        s = edge_v[b, 0, pl.ds(i * 16, 16)]
        d = edge_v[b, 1, pl.ds(i * 16, 16)]
        xs = plsc.load_gather(x0_v, [s])
        xd = plsc.load_gather(x0_v, [d])
        a0 = ax_v[b, pl.ds(i * 16, 16)]
        a1 = ay_v[b, pl.ds(i * 16, 16)]
        diff = xd - xs
        m0 = a0 != 0.0
        m1 = a1 != 0.0
        per0 = jnp.where(m0, diff / jnp.where(m0, a0, onef), zf)
        per1 = jnp.where(m1, diff / jnp.where(m1, a1, onef), zf)
        cnt0 = jnp.where(m0, onef, zf)
        cnt1 = jnp.where(m1, onef, zf)
        plsc.addupdate_scatter(acc_sx, [s], per0)
        plsc.addupdate_scatter(acc_cx, [s], cnt0)
        plsc.addupdate_scatter(acc_sy, [s], per1)
        plsc.addupdate_scatter(acc_cy, [s], cnt1)

    for k in range(SLOTS):
        b = k % 2
        wait(k, b)

        @pl.when(wid + k * NW < N_FULL)
        def _compute():
            def inner(i, c2):
                edge_group(b, i)
                return c2

            lax.fori_loop(0, CHUNK // 16, inner, 0)

        if k + 2 < SLOTS:
            issue(k + 2, b)

    @pl.when(wid == TAIL_WID)
    def _tail():
        gb = N_FULL * CHUNK
        pltpu.sync_copy(edge_hbm.at[:, pl.ds(gb, TAIL)],
                        edge_v.at[0, :, pl.ds(0, TAIL)])
        pltpu.sync_copy(ax_hbm.at[pl.ds(gb, TAIL)],
                        ax_v.at[0, pl.ds(0, TAIL)])
        pltpu.sync_copy(ay_hbm.at[pl.ds(gb, TAIL)],
                        ay_v.at[0, pl.ds(0, TAIL)])

        def inner(i, c2):
            edge_group(0, i)
            return c2

        lax.fori_loop(0, TAIL // 16, inner, 0)

    ob = wid * 4 * NODES_PAD
    pltpu.sync_copy(acc_sx, out_hbm.at[pl.ds(ob, NODES_PAD)])
    pltpu.sync_copy(acc_cx, out_hbm.at[pl.ds(ob + NODES_PAD, NODES_PAD)])
    pltpu.sync_copy(acc_sy, out_hbm.at[pl.ds(ob + 2 * NODES_PAD, NODES_PAD)])
    pltpu.sync_copy(acc_cy, out_hbm.at[pl.ds(ob + 3 * NODES_PAD, NODES_PAD)])


_sc_partials = functools.partial(
    pl.kernel,
    mesh=plsc.VectorSubcoreMesh(core_axis_name="c", subcore_axis_name="s"),
    compiler_params=pltpu.CompilerParams(needs_layout_passes=False),
    out_type=jax.ShapeDtypeStruct((NW * 4 * NODES_PAD,), jnp.float32),
    scratch_types=[
        pltpu.VMEM((X_ROWS, D_FEAT), jnp.float32),
        pltpu.VMEM_SHARED((NODES_PAD,), jnp.float32),
        pltpu.VMEM((NODES_PAD,), jnp.float32),
        pltpu.VMEM((2, 2, CHUNK), jnp.int32),
        pltpu.VMEM((2, CHUNK), jnp.float32),
        pltpu.VMEM((2, CHUNK), jnp.float32),
        pltpu.VMEM((NODES_PAD,), jnp.float32),
        pltpu.VMEM((NODES_PAD,), jnp.float32),
        pltpu.VMEM((NODES_PAD,), jnp.float32),
        pltpu.VMEM((NODES_PAD,), jnp.float32),
        pltpu.SemaphoreType.DMA,
        pltpu.SemaphoreType.DMA,
    ],
)(_sc_body)


def _tc_reduce(parts_ref, out_ref):
    p = parts_ref[...].reshape(NW, 4, NODES_PAD)
    s = jnp.sum(p, axis=0)
    dx = s[0:1, :] / jnp.maximum(s[1:2, :], 1.0)
    dy = s[2:3, :] / jnp.maximum(s[3:4, :], 1.0)
    out_ref[0:1, :] = dx
    out_ref[1:2, :] = dy


def kernel(x, edge_index, edge_attr):
    ax = edge_attr[:, 0]
    ay = edge_attr[:, 1]
    parts = _sc_partials(x, edge_index, ax, ay)
    out2 = pl.pallas_call(
        _tc_reduce,
        out_shape=jax.ShapeDtypeStruct((2, NODES_PAD), jnp.float32),
    )(parts)
    return out2[:, :N_NODES].T


# speedup vs baseline: 19.8943x; 1.0327x over previous
import functools

import jax
import jax.numpy as jnp
from jax import lax
from jax.experimental import pallas as pl
from jax.experimental.pallas import tpu as pltpu
from jax.experimental.pallas import tpu_sc as plsc

N_NODES = 10000
N_EDGES = 320000
D_FEAT = 128

NC = 2
NS = 16
NW = NC * NS
CHUNK = 2048
N_CHUNKS = -(-N_EDGES // CHUNK)
N_FULL = N_CHUNKS - 1
TAIL = N_EDGES - N_FULL * CHUNK
TAIL_WID = N_FULL % NW
SLOTS = -(-N_CHUNKS // NW)
NODES_PAD = 10240
X_ROWS = 320


def _sc_body(x0_hbm, edge_hbm, ax_hbm, ay_hbm, out_hbm,
             x0_v, edge_v, ax_v, ay_v,
             acc_sx, acc_cx, acc_sy, acc_cy, sem0, sem1, semx):
    cid = lax.axis_index("c")
    sid = lax.axis_index("s")
    wid = cid * NS + sid
    sems = (sem0, sem1)

    lanes = lax.iota(jnp.int32, 16)
    zf = jnp.zeros((16,), jnp.float32)
    onef = jnp.full((16,), 1.0, jnp.float32)
    col0 = jnp.zeros((16,), jnp.int32)

    def chunk_copies(k, b):
        gb = (wid + k * NW) * CHUNK
        return (
            pltpu.make_async_copy(edge_hbm.at[:, pl.ds(gb, CHUNK)],
                                  edge_v.at[b], sems[b]),
            pltpu.make_async_copy(ax_hbm.at[pl.ds(gb, CHUNK)],
                                  ax_v.at[b], sems[b]),
            pltpu.make_async_copy(ay_hbm.at[pl.ds(gb, CHUNK)],
                                  ay_v.at[b], sems[b]),
        )

    def issue(k, b):
        @pl.when(wid + k * NW < N_FULL)
        def _():
            for cp in chunk_copies(k, b):
                cp.start()

    def wait(k, b):
        @pl.when(wid + k * NW < N_FULL)
        def _():
            for cp in chunk_copies(k, b):
                cp.wait()

    issue(0, 0)
    issue(1, 1)
    xcp = pltpu.make_async_copy(x0_hbm, x0_v.at[pl.ds(0, N_NODES)], semx)
    xcp.start()

    def zero_body(j, carry):
        acc_sx[pl.ds(j * 16, 16)] = zf
        acc_cx[pl.ds(j * 16, 16)] = zf
        acc_sy[pl.ds(j * 16, 16)] = zf
        acc_cy[pl.ds(j * 16, 16)] = zf
        return carry

    lax.fori_loop(0, NODES_PAD // 16, zero_body, 0)
    xcp.wait()

    def edge_group(b, i):
        s = edge_v[b, 0, pl.ds(i * 16, 16)]
        d = edge_v[b, 1, pl.ds(i * 16, 16)]
        xs = plsc.load_gather(x0_v, [s])
        xd = plsc.load_gather(x0_v, [d])
        a0 = ax_v[b, pl.ds(i * 16, 16)]
        a1 = ay_v[b, pl.ds(i * 16, 16)]
        diff = xd - xs
        m0 = a0 != 0.0
        m1 = a1 != 0.0
        per0 = jnp.where(m0, diff / jnp.where(m0, a0, onef), zf)
        per1 = jnp.where(m1, diff / jnp.where(m1, a1, onef), zf)
        cnt0 = jnp.where(m0, onef, zf)
        cnt1 = jnp.where(m1, onef, zf)
        plsc.addupdate_scatter(acc_sx, [s], per0)
        plsc.addupdate_scatter(acc_cx, [s], cnt0)
        plsc.addupdate_scatter(acc_sy, [s], per1)
        plsc.addupdate_scatter(acc_cy, [s], cnt1)

    for k in range(SLOTS):
        b = k % 2
        wait(k, b)

        @pl.when(wid + k * NW < N_FULL)
        def _compute():
            def inner(i, c2):
                edge_group(b, i)
                return c2

            lax.fori_loop(0, CHUNK // 16, inner, 0)

        if k + 2 < SLOTS:
            issue(k + 2, b)

    @pl.when(wid == TAIL_WID)
    def _tail():
        gb = N_FULL * CHUNK
        pltpu.sync_copy(edge_hbm.at[:, pl.ds(gb, TAIL)],
                        edge_v.at[0, :, pl.ds(0, TAIL)])
        pltpu.sync_copy(ax_hbm.at[pl.ds(gb, TAIL)],
                        ax_v.at[0, pl.ds(0, TAIL)])
        pltpu.sync_copy(ay_hbm.at[pl.ds(gb, TAIL)],
                        ay_v.at[0, pl.ds(0, TAIL)])

        def inner(i, c2):
            edge_group(0, i)
            return c2

        lax.fori_loop(0, TAIL // 16, inner, 0)

    ob = wid * 4 * NODES_PAD
    pltpu.sync_copy(acc_sx, out_hbm.at[pl.ds(ob, NODES_PAD)])
    pltpu.sync_copy(acc_cx, out_hbm.at[pl.ds(ob + NODES_PAD, NODES_PAD)])
    pltpu.sync_copy(acc_sy, out_hbm.at[pl.ds(ob + 2 * NODES_PAD, NODES_PAD)])
    pltpu.sync_copy(acc_cy, out_hbm.at[pl.ds(ob + 3 * NODES_PAD, NODES_PAD)])


_sc_partials = functools.partial(
    pl.kernel,
    mesh=plsc.VectorSubcoreMesh(core_axis_name="c", subcore_axis_name="s"),
    compiler_params=pltpu.CompilerParams(needs_layout_passes=False),
    out_type=jax.ShapeDtypeStruct((NW * 4 * NODES_PAD,), jnp.float32),
    scratch_types=[
        pltpu.VMEM((NODES_PAD,), jnp.float32),
        pltpu.VMEM((2, 2, CHUNK), jnp.int32),
        pltpu.VMEM((2, CHUNK), jnp.float32),
        pltpu.VMEM((2, CHUNK), jnp.float32),
        pltpu.VMEM((NODES_PAD,), jnp.float32),
        pltpu.VMEM((NODES_PAD,), jnp.float32),
        pltpu.VMEM((NODES_PAD,), jnp.float32),
        pltpu.VMEM((NODES_PAD,), jnp.float32),
        pltpu.SemaphoreType.DMA,
        pltpu.SemaphoreType.DMA,
        pltpu.SemaphoreType.DMA,
    ],
)(_sc_body)


def _tc_reduce(parts_ref, out_ref):
    p = parts_ref[...].reshape(NW, 4, NODES_PAD)
    s = jnp.sum(p, axis=0)
    dx = s[0:1, :] / jnp.maximum(s[1:2, :], 1.0)
    dy = s[2:3, :] / jnp.maximum(s[3:4, :], 1.0)
    out_ref[0:1, :] = dx
    out_ref[1:2, :] = dy


def kernel(x, edge_index, edge_attr):
    x0 = x[:, 0]
    ax = edge_attr[:, 0]
    ay = edge_attr[:, 1]
    parts = _sc_partials(x0, edge_index, ax, ay)
    out2 = pl.pallas_call(
        _tc_reduce,
        out_shape=jax.ShapeDtypeStruct((2, NODES_PAD), jnp.float32),
    )(parts)
    return out2[:, :N_NODES].T
